# baseline (speedup 1.0000x reference)
import functools

import jax
import jax.numpy as jnp
from jax import lax
from jax.experimental import pallas as pl
from jax.experimental.pallas import tpu as pltpu
from jax.experimental.pallas import tpu_sc as plsc

D = 128
NC = 2
NS = 16
NW = NC * NS
CHUNK = 128
LANES = 16

_P = [32 * (j // 32) + (j % 32) // 2 + 16 * ((j % 32) % 2) for j in range(D)]


def _pack_row(flat_ref, r, out_flat_ref):
    src = pl.multiple_of(r * D, D)
    dst = pl.multiple_of(r * (D // 2), D // 2)
    for g in range(D // 32):
        a = flat_ref[pl.ds(src + 32 * g, LANES)]
        b = flat_ref[pl.ds(src + 32 * g + LANES, LANES)]
        packed = plsc.pack(a, b, format=plsc.PackFormat.INTERLEAVED)
        out_flat_ref[pl.ds(dst + LANES * g, LANES)] = plsc.bitcast(
            packed, jnp.float32
        )


def _pack_chunk(rows2d, out_chunk_bf):
    out_f32 = out_chunk_bf

    def body(r, _):
        for g in range(D // 32):
            a = rows2d[r, pl.ds(32 * g, LANES)]
            b = rows2d[r, pl.ds(32 * g + LANES, LANES)]
            out_f32[r, pl.ds(32 * g, LANES)] = a.astype(jnp.bfloat16)
            out_f32[r, pl.ds(32 * g + LANES, LANES)] = b.astype(jnp.bfloat16)
        return 0

    lax.fori_loop(0, CHUNK, body, 0)


def _gather_body(n_chunk, table_hbm, idx_hbm, out_hbm, idx_v, rows_v,
                 out_bf, sem0, sem1, out_sem):
    wid = lax.axis_index("s") * NC + lax.axis_index("c")
    pltpu.sync_copy(idx_hbm.at[wid], idx_v)
    sems = [sem0, sem1]

    def fire(j):
        return pltpu.async_copy(
            table_hbm.at[idx_v.at[j]], rows_v.at[j % 2], sems[j % 2]
        )

    inflight = {0: fire(0)}
    if n_chunk > 1:
        inflight[1] = fire(1)
    for j in range(n_chunk):
        inflight.pop(j).wait()
        _pack_chunk(rows_v.at[j % 2], out_bf.at[j])
        if j + 2 < n_chunk:
            inflight[j + 2] = fire(j + 2)
    pltpu.async_copy(out_bf, out_hbm.at[wid], out_sem).wait()


def _sc_gather_bf16(table, idx3):
    n_chunk = idx3.shape[1]
    mesh = plsc.VectorSubcoreMesh(
        core_axis_name="c", subcore_axis_name="s", num_cores=NC, num_subcores=NS
    )
    return pl.kernel(
        functools.partial(_gather_body, n_chunk),
        out_type=jax.ShapeDtypeStruct((NW, n_chunk, CHUNK, D), jnp.bfloat16),
        mesh=mesh,
        compiler_params=pltpu.CompilerParams(use_tc_tiling_on_sc=False),
        scratch_types=[
            pltpu.VMEM((n_chunk, CHUNK), jnp.int32),
            pltpu.VMEM((2, CHUNK, D), jnp.float32),
            pltpu.VMEM((n_chunk, CHUNK, D), jnp.bfloat16),
            pltpu.SemaphoreType.DMA,
            pltpu.SemaphoreType.DMA,
            pltpu.SemaphoreType.DMA,
        ],
    )(table, idx3)


def _silu_mm_body(h_ref, w_ref, b_ref, o_ref):
    h = h_ref[...].astype(jnp.float32)
    h = h * jax.nn.sigmoid(h)
    o_ref[...] = (
        lax.dot_general(h, w_ref[...], (((1,), (1,)), ((), ())),
                        preferred_element_type=jnp.float32)
        + b_ref[...]
    )


def _tc_silu_mm(gathered, Wp, b2, block):
    batch = gathered.shape[0]
    grid = (batch // block,)
    return pl.pallas_call(
        _silu_mm_body,
        out_shape=jax.ShapeDtypeStruct((batch, D), jnp.float32),
        grid=grid,
        in_specs=[
            pl.BlockSpec((block, D), lambda i: (i, 0)),
            pl.BlockSpec((D, D), lambda i: (0, 0)),
            pl.BlockSpec((1, D), lambda i: (0, 0)),
        ],
        out_specs=pl.BlockSpec((block, D), lambda i: (i, 0)),
    )(gathered, Wp, b2)


def kernel(x, emb_table, W, b):
    batch = x.shape[0]
    n_chunk = batch // (NW * CHUNK)
    idx3 = x.reshape(NW, n_chunk, CHUNK)
    gathered = _sc_gather_bf16(emb_table, idx3).reshape(batch, D)
    return _tc_silu_mm(gathered, W, b.reshape(1, D), block=1024)


# speedup vs baseline: 1.1040x; 1.1040x over previous
import functools

import jax
import jax.numpy as jnp
from jax import lax
from jax.experimental import pallas as pl
from jax.experimental.pallas import tpu as pltpu
from jax.experimental.pallas import tpu_sc as plsc

D = 128
NC = 2
NS = 16
NW = NC * NS
CHUNK = 128
LANES = 16

_P = [32 * (j // 32) + (j % 32) // 2 + 16 * ((j % 32) % 2) for j in range(D)]


def _pack_row(flat_ref, r, out_flat_ref):
    src = pl.multiple_of(r * D, D)
    dst = pl.multiple_of(r * (D // 2), D // 2)
    for g in range(D // 32):
        a = flat_ref[pl.ds(src + 32 * g, LANES)]
        b = flat_ref[pl.ds(src + 32 * g + LANES, LANES)]
        packed = plsc.pack(a, b, format=plsc.PackFormat.INTERLEAVED)
        out_flat_ref[pl.ds(dst + LANES * g, LANES)] = plsc.bitcast(
            packed, jnp.float32
        )


def _pack_chunk(rows2d, out_chunk_bf):
    @plsc.parallel_loop(0, CHUNK, unroll=4)
    def _cvt(r):
        for g in range(D // LANES):
            v = rows2d[r, pl.ds(LANES * g, LANES)]
            out_chunk_bf[r, pl.ds(LANES * g, LANES)] = v.astype(jnp.bfloat16)


def _gather_body(n_chunk, table_hbm, idx_hbm, out_hbm, idx_v, rows_v,
                 out_bf, sem0, sem1, out_sem):
    wid = lax.axis_index("s") * NC + lax.axis_index("c")
    pltpu.sync_copy(idx_hbm.at[wid], idx_v)
    sems = [sem0, sem1]

    def fire(j):
        return pltpu.async_copy(
            table_hbm.at[idx_v.at[j]], rows_v.at[j % 2], sems[j % 2]
        )

    inflight = {0: fire(0)}
    if n_chunk > 1:
        inflight[1] = fire(1)
    for j in range(n_chunk):
        inflight.pop(j).wait()
        _pack_chunk(rows_v.at[j % 2], out_bf.at[j])
        if j + 2 < n_chunk:
            inflight[j + 2] = fire(j + 2)
    pltpu.async_copy(out_bf, out_hbm.at[wid], out_sem).wait()


def _sc_gather_bf16(table, idx3):
    n_chunk = idx3.shape[1]
    mesh = plsc.VectorSubcoreMesh(
        core_axis_name="c", subcore_axis_name="s", num_cores=NC, num_subcores=NS
    )
    return pl.kernel(
        functools.partial(_gather_body, n_chunk),
        out_type=jax.ShapeDtypeStruct((NW, n_chunk, CHUNK, D), jnp.bfloat16),
        mesh=mesh,
        compiler_params=pltpu.CompilerParams(use_tc_tiling_on_sc=False),
        scratch_types=[
            pltpu.VMEM((n_chunk, CHUNK), jnp.int32),
            pltpu.VMEM((2, CHUNK, D), jnp.float32),
            pltpu.VMEM((n_chunk, CHUNK, D), jnp.bfloat16),
            pltpu.SemaphoreType.DMA,
            pltpu.SemaphoreType.DMA,
            pltpu.SemaphoreType.DMA,
        ],
    )(table, idx3)


def _silu_mm_body(h_ref, w_ref, b_ref, o_ref):
    h = h_ref[...].astype(jnp.float32)
    h = h * jax.nn.sigmoid(h)
    o_ref[...] = (
        lax.dot_general(h, w_ref[...], (((1,), (1,)), ((), ())),
                        preferred_element_type=jnp.float32)
        + b_ref[...]
    )


def _tc_silu_mm(gathered, Wp, b2, block):
    batch = gathered.shape[0]
    grid = (batch // block,)
    return pl.pallas_call(
        _silu_mm_body,
        out_shape=jax.ShapeDtypeStruct((batch, D), jnp.float32),
        grid=grid,
        in_specs=[
            pl.BlockSpec((block, D), lambda i: (i, 0)),
            pl.BlockSpec((D, D), lambda i: (0, 0)),
            pl.BlockSpec((1, D), lambda i: (0, 0)),
        ],
        out_specs=pl.BlockSpec((block, D), lambda i: (i, 0)),
    )(gathered, Wp, b2)


def kernel(x, emb_table, W, b):
    batch = x.shape[0]
    n_chunk = batch // (NW * CHUNK)
    idx3 = x.reshape(NW, n_chunk, CHUNK)
    gathered = _sc_gather_bf16(emb_table, idx3).reshape(batch, D)
    return _tc_silu_mm(gathered, W, b.reshape(1, D), block=1024)


# speedup vs baseline: 1.7648x; 1.5986x over previous
import functools

import jax
import jax.numpy as jnp
from jax import lax
from jax.experimental import pallas as pl
from jax.experimental.pallas import tpu as pltpu
from jax.experimental.pallas import tpu_sc as plsc

D = 128
NC = 2
NS = 16
NW = NC * NS
CHUNK = 128
BLOCK = 2048


def _gather_body(n_chunk, table_hbm, idx_hbm, out_hbm, idx_v, rows_v,
                 g_sem0, g_sem1, w_sem0, w_sem1):
    wid = lax.axis_index("s") * NC + lax.axis_index("c")
    pltpu.sync_copy(idx_hbm.at[wid], idx_v)
    g_sems = [g_sem0, g_sem1]
    w_sems = [w_sem0, w_sem1]

    def fire(j):
        return pltpu.async_copy(
            table_hbm.at[idx_v.at[j]], rows_v.at[j % 2], g_sems[j % 2]
        )

    inflight = {0: fire(0)}
    if n_chunk > 1:
        inflight[1] = fire(1)
    writes = {}
    for j in range(n_chunk):
        inflight.pop(j).wait()
        writes[j] = pltpu.async_copy(
            rows_v.at[j % 2], out_hbm.at[wid].at[j], w_sems[j % 2]
        )
        if j + 2 < n_chunk:
            writes.pop(j).wait()
            inflight[j + 2] = fire(j + 2)
    for w in writes.values():
        w.wait()


def _sc_gather(table, idx3):
    n_chunk = idx3.shape[1]
    mesh = plsc.VectorSubcoreMesh(
        core_axis_name="c", subcore_axis_name="s", num_cores=NC, num_subcores=NS
    )
    return pl.kernel(
        functools.partial(_gather_body, n_chunk),
        out_type=jax.ShapeDtypeStruct((NW, n_chunk, CHUNK, D), jnp.float32),
        mesh=mesh,
        scratch_types=[
            pltpu.VMEM((n_chunk, CHUNK), jnp.int32),
            pltpu.VMEM((2, CHUNK, D), jnp.float32),
            pltpu.SemaphoreType.DMA,
            pltpu.SemaphoreType.DMA,
            pltpu.SemaphoreType.DMA,
            pltpu.SemaphoreType.DMA,
        ],
    )(table, idx3)


def _silu_mm_body(h_ref, w_ref, b_ref, o_ref):
    h = h_ref[...]
    h = h * jax.nn.sigmoid(h)
    o_ref[...] = (
        lax.dot_general(h, w_ref[...], (((1,), (1,)), ((), ())),
                        preferred_element_type=jnp.float32)
        + b_ref[...]
    )


def _tc_first(gathered, W, b2, total_batch):
    half = gathered.shape[0]
    return pl.pallas_call(
        _silu_mm_body,
        out_shape=jax.ShapeDtypeStruct((total_batch, D), jnp.float32),
        grid=(half // BLOCK,),
        in_specs=[
            pl.BlockSpec((BLOCK, D), lambda i: (i, 0)),
            pl.BlockSpec((D, D), lambda i: (0, 0)),
            pl.BlockSpec((1, D), lambda i: (0, 0)),
        ],
        out_specs=pl.BlockSpec((BLOCK, D), lambda i: (i, 0)),
    )(gathered, W, b2)


def _silu_mm_body2(h_ref, w_ref, b_ref, y_ref, o_ref):
    del y_ref
    _silu_mm_body(h_ref, w_ref, b_ref, o_ref)


def _tc_second(gathered, W, b2, y):
    half = gathered.shape[0]
    nb2 = half // BLOCK
    return pl.pallas_call(
        _silu_mm_body2,
        out_shape=jax.ShapeDtypeStruct(y.shape, jnp.float32),
        grid=(nb2,),
        in_specs=[
            pl.BlockSpec((BLOCK, D), lambda i: (i, 0)),
            pl.BlockSpec((D, D), lambda i: (0, 0)),
            pl.BlockSpec((1, D), lambda i: (0, 0)),
            pl.BlockSpec(memory_space=pl.ANY),
        ],
        out_specs=pl.BlockSpec((BLOCK, D), lambda i: (i + nb2, 0)),
        input_output_aliases={3: 0},
    )(gathered, W, b2, y)


def kernel(x, emb_table, W, b):
    batch = x.shape[0]
    half = batch // 2
    n_chunk = half // (NW * CHUNK)
    idx = x.reshape(2, NW, n_chunk, CHUNK)
    b2 = b.reshape(1, D)
    g0 = _sc_gather(emb_table, idx[0]).reshape(half, D)
    g1 = _sc_gather(emb_table, idx[1]).reshape(half, D)
    y = _tc_first(g0, W, b2, batch)
    return _tc_second(g1, W, b2, y)


# speedup vs baseline: 1.7751x; 1.0058x over previous
import functools

import jax
import jax.numpy as jnp
from jax import lax
from jax.experimental import pallas as pl
from jax.experimental.pallas import tpu as pltpu
from jax.experimental.pallas import tpu_sc as plsc

D = 128
NC = 2
NS = 16
NW = NC * NS
CHUNK = 128
BLOCK = 2048


def _gather_body(n_chunk, h, table_hbm, idx_hbm, out_hbm, idx_v, rows_v,
                 g_sem0, g_sem1, w_sem0, w_sem1):
    wid = lax.axis_index("s") * NC + lax.axis_index("c")
    pltpu.sync_copy(idx_hbm.at[h].at[wid], idx_v)
    g_sems = [g_sem0, g_sem1]
    w_sems = [w_sem0, w_sem1]

    def fire(j):
        return pltpu.async_copy(
            table_hbm.at[idx_v.at[j]], rows_v.at[j % 2], g_sems[j % 2]
        )

    inflight = {0: fire(0)}
    if n_chunk > 1:
        inflight[1] = fire(1)
    writes = {}
    for j in range(n_chunk):
        inflight.pop(j).wait()
        writes[j] = pltpu.async_copy(
            rows_v.at[j % 2], out_hbm.at[wid].at[j], w_sems[j % 2]
        )
        if j + 2 < n_chunk:
            writes.pop(j).wait()
            inflight[j + 2] = fire(j + 2)
    for w in writes.values():
        w.wait()


def _sc_gather(table, idx4, h):
    n_chunk = idx4.shape[2]
    mesh = plsc.VectorSubcoreMesh(
        core_axis_name="c", subcore_axis_name="s", num_cores=NC, num_subcores=NS
    )
    return pl.kernel(
        functools.partial(_gather_body, n_chunk, h),
        out_type=jax.ShapeDtypeStruct((NW, n_chunk, CHUNK, D), jnp.float32),
        mesh=mesh,
        scratch_types=[
            pltpu.VMEM((n_chunk, CHUNK), jnp.int32),
            pltpu.VMEM((2, CHUNK, D), jnp.float32),
            pltpu.SemaphoreType.DMA,
            pltpu.SemaphoreType.DMA,
            pltpu.SemaphoreType.DMA,
            pltpu.SemaphoreType.DMA,
        ],
    )(table, idx4)


def _silu_mm_body(h_ref, w_ref, b_ref, o_ref):
    h = h_ref[...]
    h = h * jax.nn.sigmoid(h)
    o_ref[...] = (
        lax.dot_general(h, w_ref[...], (((1,), (1,)), ((), ())),
                        preferred_element_type=jnp.float32)
        + b_ref[...]
    )


def _tc_first(gathered, W, b2, total_batch):
    half = gathered.shape[0]
    return pl.pallas_call(
        _silu_mm_body,
        out_shape=jax.ShapeDtypeStruct((total_batch, D), jnp.float32),
        grid=(half // BLOCK,),
        in_specs=[
            pl.BlockSpec((BLOCK, D), lambda i: (i, 0)),
            pl.BlockSpec((D, D), lambda i: (0, 0)),
            pl.BlockSpec((1, D), lambda i: (0, 0)),
        ],
        out_specs=pl.BlockSpec((BLOCK, D), lambda i: (i, 0)),
    )(gathered, W, b2)


def _silu_mm_body2(h_ref, w_ref, b_ref, y_ref, o_ref):
    del y_ref
    _silu_mm_body(h_ref, w_ref, b_ref, o_ref)


def _tc_second(gathered, W, b2, y):
    half = gathered.shape[0]
    nb2 = half // BLOCK
    return pl.pallas_call(
        _silu_mm_body2,
        out_shape=jax.ShapeDtypeStruct(y.shape, jnp.float32),
        grid=(nb2,),
        in_specs=[
            pl.BlockSpec((BLOCK, D), lambda i: (i, 0)),
            pl.BlockSpec((D, D), lambda i: (0, 0)),
            pl.BlockSpec((1, D), lambda i: (0, 0)),
            pl.BlockSpec(memory_space=pl.ANY),
        ],
        out_specs=pl.BlockSpec((BLOCK, D), lambda i: (i + nb2, 0)),
        input_output_aliases={3: 0},
    )(gathered, W, b2, y)


def kernel(x, emb_table, W, b):
    batch = x.shape[0]
    half = batch // 2
    n_chunk = half // (NW * CHUNK)
    idx = x.reshape(2, NW, n_chunk, CHUNK)
    b2 = b.reshape(1, D)
    g0 = _sc_gather(emb_table, idx, 0).reshape(half, D)
    g1 = _sc_gather(emb_table, idx, 1).reshape(half, D)
    y = _tc_first(g0, W, b2, batch)
    return _tc_second(g1, W, b2, y)
